# baseline (speedup 1.0000x reference)
import math
import functools
import jax
import jax.numpy as jnp
from jax import lax
from jax.experimental import pallas as pl
from jax.experimental.pallas import tpu as pltpu
from jax.experimental.pallas import tpu_sc as plsc

N_PAPER = 10000
N_AUTHOR = 10000
E_EDGES = 160000
IN_DIM = 256
OUT_DIM = 256
N_HEADS = 8
D_K = OUT_DIM // N_HEADS
NTILE = 32
CHUNK = 64
NGROUP = E_EDGES // CHUNK
NPAD = 10240
NODES_PER_TILE = NPAD // 16


def _mm_bias_kernel(x_ref, w_ref, b_ref, o_ref):
    o_ref[...] = jnp.dot(x_ref[...], w_ref[...],
                         preferred_element_type=jnp.float32) + b_ref[...]


def _mm_bias(x, w, b, block_rows=2000):
    n = x.shape[0]
    return pl.pallas_call(
        _mm_bias_kernel,
        grid=(n // block_rows,),
        in_specs=[
            pl.BlockSpec((block_rows, x.shape[1]), lambda i: (i, 0)),
            pl.BlockSpec((w.shape[0], w.shape[1]), lambda i: (0, 0)),
            pl.BlockSpec((1, w.shape[1]), lambda i: (0, 0)),
        ],
        out_specs=pl.BlockSpec((block_rows, w.shape[1]), lambda i: (i, 0)),
        out_shape=jax.ShapeDtypeStruct((n, w.shape[1]), jnp.float32),
    )(x, w, b.reshape(1, -1))


def _final_kernel(a0_ref, a1_ref, hp_ref, ha_ref, wa0_ref, ba0_ref, ba1_ref,
                  sk_ref, op_ref, oa_ref):
    alpha0 = sk_ref[0, 0]
    alpha1 = sk_ref[0, 1]
    agg = a0_ref[...] + a1_ref[...]
    t = jnp.dot(agg, wa0_ref[...],
                preferred_element_type=jnp.float32) + ba0_ref[...]
    op_ref[...] = t * alpha0 + hp_ref[...] * (1.0 - alpha0)
    oa_ref[...] = ba1_ref[...] * alpha1 + ha_ref[...] * (1.0 - alpha1)


def _final(agg0, agg1, h_paper, h_author, Wa0h, ba0, ba1, skip,
           block_rows=2000):
    n = N_PAPER
    alphas = jax.nn.sigmoid(skip).reshape(1, 2)
    return pl.pallas_call(
        _final_kernel,
        grid=(n // block_rows,),
        in_specs=[
            pl.BlockSpec((block_rows, OUT_DIM), lambda i: (i, 0)),
            pl.BlockSpec((block_rows, OUT_DIM), lambda i: (i, 0)),
            pl.BlockSpec((block_rows, IN_DIM), lambda i: (i, 0)),
            pl.BlockSpec((block_rows, IN_DIM), lambda i: (i, 0)),
            pl.BlockSpec((OUT_DIM, OUT_DIM), lambda i: (0, 0)),
            pl.BlockSpec((1, OUT_DIM), lambda i: (0, 0)),
            pl.BlockSpec((1, OUT_DIM), lambda i: (0, 0)),
            pl.BlockSpec((1, 2), lambda i: (0, 0)),
        ],
        out_specs=[
            pl.BlockSpec((block_rows, OUT_DIM), lambda i: (i, 0)),
            pl.BlockSpec((block_rows, OUT_DIM), lambda i: (i, 0)),
        ],
        out_shape=[
            jax.ShapeDtypeStruct((n, OUT_DIM), jnp.float32),
            jax.ShapeDtypeStruct((n, OUT_DIM), jnp.float32),
        ],
    )(agg0, agg1, h_paper, h_author, Wa0h, ba0.reshape(1, -1),
      ba1.reshape(1, -1), alphas)


def _fold(W, b, rel):
    Wf = jnp.einsum('ihj,hjk->ihk', W.reshape(IN_DIM, N_HEADS, D_K),
                    rel).reshape(IN_DIM, OUT_DIM)
    bf = jnp.einsum('hj,hjk->hk', b.reshape(N_HEADS, D_K), rel).reshape(OUT_DIM)
    return Wf, bf


_SC_MESH = plsc.VectorSubcoreMesh(core_axis_name="c", subcore_axis_name="s")


@functools.partial(
    pl.kernel,
    out_type=[
        jax.ShapeDtypeStruct((E_EDGES, N_HEADS), jnp.float32),
        jax.ShapeDtypeStruct((E_EDGES, N_HEADS), jnp.float32),
        jax.ShapeDtypeStruct((2, 2, NPAD, N_HEADS), jnp.float32),
    ],
    mesh=_SC_MESH,
    compiler_params=pltpu.CompilerParams(use_tc_tiling_on_sc=False,
                                         needs_layout_passes=False),
    scratch_types=[
        pltpu.VMEM((CHUNK,), jnp.int32),
        pltpu.VMEM((CHUNK,), jnp.int32),
        pltpu.VMEM((CHUNK, IN_DIM), jnp.float32),
        pltpu.VMEM((CHUNK, IN_DIM), jnp.float32),
        pltpu.VMEM((CHUNK, N_HEADS), jnp.float32),
        pltpu.VMEM_SHARED((NPAD, N_HEADS), jnp.float32),
    ],
)
def _scores_sc(k0_hbm, q_hbm, k1_hbm, eg0_hbm, eg1_hbm, z8_hbm,
               e0_hbm, e1_hbm, den_hbm,
               srcb, dstb, kbuf, qbuf, ebuf, den_sh):
    c = lax.axis_index("c")
    s = lax.axis_index("s")
    w = s * 2 + c
    iotas = [lax.iota(jnp.int32, 16) + 16 * b for b in range(CHUNK // 16)]
    g_lo = lax.shift_right_logical(NGROUP // 4 * w, 3)
    g_hi = lax.shift_right_logical(NGROUP // 4 * (w + 1), 3)
    nbase = NODES_PER_TILE * s

    for ridx, (eg_hbm, k_hbm, e_hbm) in enumerate(
            [(eg0_hbm, k0_hbm, e0_hbm), (eg1_hbm, k1_hbm, e1_hbm)]):
        pltpu.sync_copy(z8_hbm.at[pl.ds(nbase, NODES_PER_TILE)],
                        den_sh.at[pl.ds(nbase, NODES_PER_TILE)])
        plsc.subcore_barrier()

        @pl.loop(g_lo, g_hi)
        def _group(g):
            eb = g * CHUNK
            pltpu.sync_copy(eg_hbm.at[pl.ds(eb, CHUNK)], srcb)
            pltpu.sync_copy(eg_hbm.at[pl.ds(E_EDGES + eb, CHUNK)], dstb)
            pltpu.sync_copy(k_hbm.at[srcb], kbuf)
            pltpu.sync_copy(q_hbm.at[dstb], qbuf)
            for b in range(CHUNK // 16):
                lanes = iotas[b]
                for h in range(N_HEADS):
                    acc = jnp.zeros((16,), jnp.float32)
                    for d2 in range(D_K):
                        d = jnp.full((16,), D_K * h + d2, jnp.int32)
                        qc = plsc.load_gather(qbuf, [lanes, d])
                        kc = plsc.load_gather(kbuf, [lanes, d])
                        acc = acc + qc * kc
                    eh = jnp.exp(acc)
                    plsc.store_scatter(
                        ebuf, [lanes, jnp.full((16,), h, jnp.int32)], eh)
            pltpu.sync_copy(ebuf, e_hbm.at[pl.ds(eb, CHUNK)])
            pltpu.sync_copy(ebuf, den_sh.at[dstb], add=True)

        plsc.subcore_barrier()
        pltpu.sync_copy(den_sh.at[pl.ds(nbase, NODES_PER_TILE)],
                        den_hbm.at[c, ridx, pl.ds(nbase, NODES_PER_TILE)])
        plsc.subcore_barrier()


def kernel(h_paper, h_author, edge_writes, edge_cites, Wk, bk, Wv, bv, Wq, bq,
           Wa, ba, rel_att, rel_msg, rel_pri, skip):
    sqrt_dk = math.sqrt(D_K)
    att0 = rel_att[0] * (rel_pri[0][:, None, None] / sqrt_dk)
    att1 = rel_att[1] * (rel_pri[1][:, None, None] / sqrt_dk)
    Wk0, bk0 = _fold(Wk[1], bk[1], att0)
    Wv0, bv0 = _fold(Wv[1], bv[1], rel_msg[0])
    Wk1, bk1 = _fold(Wk[0], bk[0], att1)
    Wv1, bv1 = _fold(Wv[0], bv[0], rel_msg[1])

    K0 = _mm_bias(h_author, Wk0, bk0)
    V0 = _mm_bias(h_author, Wv0, bv0)
    K1 = _mm_bias(h_paper, Wk1, bk1)
    V1 = _mm_bias(h_paper, Wv1, bv1)
    Q = _mm_bias(h_paper, Wq[0], bq[0])

    z8 = jnp.zeros((NPAD, N_HEADS), jnp.float32)
    e0, e1, den = _scores_sc(K0, Q, K1, edge_writes.reshape(-1),
                             edge_cites.reshape(-1), z8)
    den = den[:, :, :N_PAPER, :]

    def msg_phase(V, edges, e, den_r):
        src, dst = edges[0], edges[1]
        msg = V[src].reshape(-1, N_HEADS, D_K) * e[:, :, None]
        num = jax.ops.segment_sum(msg, dst, num_segments=N_PAPER)
        return num / (den_r[:, :, None] + 1e-9)

    agg0 = msg_phase(V0, edge_writes, e0, den[0, 0] + den[1, 0])
    agg1 = msg_phase(V1, edge_cites, e1, den[0, 1] + den[1, 1])
    agg0 = agg0.reshape(N_PAPER, OUT_DIM)
    agg1 = agg1.reshape(N_PAPER, OUT_DIM)

    out_p, out_a = _final(agg0, agg1, h_paper, h_author, 0.5 * Wa[0],
                          ba[0], ba[1], skip)
    return (out_p, out_a)


# speedup vs baseline: 7.9234x; 7.9234x over previous
import math
import functools
import jax
import jax.numpy as jnp
from jax import lax
from jax.experimental import pallas as pl
from jax.experimental.pallas import tpu as pltpu
from jax.experimental.pallas import tpu_sc as plsc

N_PAPER = 10000
N_AUTHOR = 10000
E_EDGES = 160000
IN_DIM = 256
OUT_DIM = 256
N_HEADS = 8
D_K = OUT_DIM // N_HEADS
NTILE = 32
CHUNK = 64
NGROUP = E_EDGES // CHUNK
NPAD = 10240
NODES_PER_TILE = NPAD // 16


def _mm_bias_kernel(x_ref, w_ref, b_ref, o_ref):
    o_ref[...] = jnp.dot(x_ref[...], w_ref[...],
                         preferred_element_type=jnp.float32) + b_ref[...]


def _mm_bias(x, w, b, block_rows=2000):
    n = x.shape[0]
    return pl.pallas_call(
        _mm_bias_kernel,
        grid=(n // block_rows,),
        in_specs=[
            pl.BlockSpec((block_rows, x.shape[1]), lambda i: (i, 0)),
            pl.BlockSpec((w.shape[0], w.shape[1]), lambda i: (0, 0)),
            pl.BlockSpec((1, w.shape[1]), lambda i: (0, 0)),
        ],
        out_specs=pl.BlockSpec((block_rows, w.shape[1]), lambda i: (i, 0)),
        out_shape=jax.ShapeDtypeStruct((n, w.shape[1]), jnp.float32),
    )(x, w, b.reshape(1, -1))


def _final_kernel(a0_ref, a1_ref, hp_ref, ha_ref, wa0_ref, ba0_ref, ba1_ref,
                  sk_ref, op_ref, oa_ref):
    alpha0 = sk_ref[0, 0]
    alpha1 = sk_ref[0, 1]
    agg = a0_ref[...] + a1_ref[...]
    t = jnp.dot(agg, wa0_ref[...],
                preferred_element_type=jnp.float32) + ba0_ref[...]
    op_ref[...] = t * alpha0 + hp_ref[...] * (1.0 - alpha0)
    oa_ref[...] = ba1_ref[...] * alpha1 + ha_ref[...] * (1.0 - alpha1)


def _final(agg0, agg1, h_paper, h_author, Wa0h, ba0, ba1, skip,
           block_rows=2000):
    n = N_PAPER
    alphas = jax.nn.sigmoid(skip).reshape(1, 2)
    return pl.pallas_call(
        _final_kernel,
        grid=(n // block_rows,),
        in_specs=[
            pl.BlockSpec((block_rows, OUT_DIM), lambda i: (i, 0)),
            pl.BlockSpec((block_rows, OUT_DIM), lambda i: (i, 0)),
            pl.BlockSpec((block_rows, IN_DIM), lambda i: (i, 0)),
            pl.BlockSpec((block_rows, IN_DIM), lambda i: (i, 0)),
            pl.BlockSpec((OUT_DIM, OUT_DIM), lambda i: (0, 0)),
            pl.BlockSpec((1, OUT_DIM), lambda i: (0, 0)),
            pl.BlockSpec((1, OUT_DIM), lambda i: (0, 0)),
            pl.BlockSpec((1, 2), lambda i: (0, 0)),
        ],
        out_specs=[
            pl.BlockSpec((block_rows, OUT_DIM), lambda i: (i, 0)),
            pl.BlockSpec((block_rows, OUT_DIM), lambda i: (i, 0)),
        ],
        out_shape=[
            jax.ShapeDtypeStruct((n, OUT_DIM), jnp.float32),
            jax.ShapeDtypeStruct((n, OUT_DIM), jnp.float32),
        ],
    )(agg0, agg1, h_paper, h_author, Wa0h, ba0.reshape(1, -1),
      ba1.reshape(1, -1), alphas)


def _fold(W, b, rel):
    Wf = jnp.einsum('ihj,hjk->ihk', W.reshape(IN_DIM, N_HEADS, D_K),
                    rel).reshape(IN_DIM, OUT_DIM)
    bf = jnp.einsum('hj,hjk->hk', b.reshape(N_HEADS, D_K), rel).reshape(OUT_DIM)
    return Wf, bf


_SC_MESH = plsc.VectorSubcoreMesh(core_axis_name="c", subcore_axis_name="s")


@functools.partial(
    pl.kernel,
    out_type=[
        jax.ShapeDtypeStruct((E_EDGES, N_HEADS), jnp.float32),
        jax.ShapeDtypeStruct((E_EDGES, N_HEADS), jnp.float32),
        jax.ShapeDtypeStruct((2, 2, NPAD, N_HEADS), jnp.float32),
    ],
    mesh=_SC_MESH,
    compiler_params=pltpu.CompilerParams(use_tc_tiling_on_sc=False,
                                         needs_layout_passes=False),
    scratch_types=[
        pltpu.VMEM((CHUNK,), jnp.int32),
        pltpu.VMEM((CHUNK,), jnp.int32),
        pltpu.VMEM((CHUNK, IN_DIM), jnp.float32),
        pltpu.VMEM((CHUNK, IN_DIM), jnp.float32),
        pltpu.VMEM((CHUNK, N_HEADS), jnp.float32),
        pltpu.VMEM_SHARED((NPAD, N_HEADS), jnp.float32),
    ],
)
def _scores_sc(k0_hbm, q_hbm, k1_hbm, eg0_hbm, eg1_hbm, z8_hbm,
               e0_hbm, e1_hbm, den_hbm,
               srcb, dstb, kbuf, qbuf, ebuf, den_sh):
    c = lax.axis_index("c")
    s = lax.axis_index("s")
    w = s * 2 + c
    iotas = [lax.iota(jnp.int32, 16) + 16 * b for b in range(CHUNK // 16)]
    g_lo = lax.shift_right_logical(NGROUP // 4 * w, 3)
    g_hi = lax.shift_right_logical(NGROUP // 4 * (w + 1), 3)
    nbase = NODES_PER_TILE * s

    for ridx, (eg_hbm, k_hbm, e_hbm) in enumerate(
            [(eg0_hbm, k0_hbm, e0_hbm), (eg1_hbm, k1_hbm, e1_hbm)]):
        pltpu.sync_copy(z8_hbm.at[pl.ds(nbase, NODES_PER_TILE)],
                        den_sh.at[pl.ds(nbase, NODES_PER_TILE)])
        plsc.subcore_barrier()

        @pl.loop(g_lo, g_hi)
        def _group(g):
            eb = g * CHUNK
            pltpu.sync_copy(eg_hbm.at[pl.ds(eb, CHUNK)], srcb)
            pltpu.sync_copy(eg_hbm.at[pl.ds(E_EDGES + eb, CHUNK)], dstb)
            pltpu.sync_copy(k_hbm.at[srcb], kbuf)
            pltpu.sync_copy(q_hbm.at[dstb], qbuf)
            for b in range(CHUNK // 16):
                lanes = iotas[b]
                for h in range(N_HEADS):
                    acc = jnp.zeros((16,), jnp.float32)
                    for d2 in range(D_K):
                        d = jnp.full((16,), D_K * h + d2, jnp.int32)
                        qc = plsc.load_gather(qbuf, [lanes, d])
                        kc = plsc.load_gather(kbuf, [lanes, d])
                        acc = acc + qc * kc
                    eh = jnp.exp(acc)
                    plsc.store_scatter(
                        ebuf, [lanes, jnp.full((16,), h, jnp.int32)], eh)
            pltpu.sync_copy(ebuf, e_hbm.at[pl.ds(eb, CHUNK)])
            pltpu.sync_copy(ebuf, den_sh.at[dstb], add=True)

        plsc.subcore_barrier()
        pltpu.sync_copy(den_sh.at[pl.ds(nbase, NODES_PER_TILE)],
                        den_hbm.at[c, ridx, pl.ds(nbase, NODES_PER_TILE)])
        plsc.subcore_barrier()


HALF = 128


@functools.partial(
    pl.kernel,
    out_type=[
        jax.ShapeDtypeStruct((2, NPAD, HALF), jnp.float32),
        jax.ShapeDtypeStruct((2, NPAD, HALF), jnp.float32),
    ],
    mesh=_SC_MESH,
    compiler_params=pltpu.CompilerParams(use_tc_tiling_on_sc=False,
                                         needs_layout_passes=False),
    scratch_types=[
        pltpu.VMEM((CHUNK,), jnp.int32),
        pltpu.VMEM((CHUNK,), jnp.int32),
        pltpu.VMEM((CHUNK,), jnp.int32),
        pltpu.VMEM((CHUNK, HALF), jnp.float32),
        pltpu.VMEM((CHUNK, N_HEADS), jnp.float32),
        pltpu.VMEM((CHUNK, HALF), jnp.float32),
        pltpu.VMEM((CHUNK, HALF), jnp.float32),
        pltpu.VMEM((CHUNK, N_HEADS), jnp.float32),
        pltpu.VMEM((CHUNK, N_HEADS), jnp.float32),
        pltpu.VMEM_SHARED((NPAD, HALF), jnp.float32),
    ],
)
def _messages_sc(eg0_hbm, eg1_hbm, e0_hbm, e1_hbm, v0_hbm, v1_hbm, den_hbm,
                 zbig_hbm, agg0_hbm, agg1_hbm,
                 srcb, dstb, vidxb, vbuf, ebuf, mbuf, nbuf, d0b, d1b, num_sh):
    c = lax.axis_index("c")
    s = lax.axis_index("s")
    g_lo = lax.shift_right_logical(NGROUP // 4 * s, 2)
    g_hi = lax.shift_right_logical(NGROUP // 4 * (s + 1), 2)
    nbase = NODES_PER_TILE * s
    base_h = 4 * c

    for ridx, (eg_hbm, e_hbm, v_hbm, agg_hbm) in enumerate(
            [(eg0_hbm, e0_hbm, v0_hbm, agg0_hbm),
             (eg1_hbm, e1_hbm, v1_hbm, agg1_hbm)]):
        pltpu.sync_copy(zbig_hbm.at[pl.ds(nbase, NODES_PER_TILE)],
                        num_sh.at[pl.ds(nbase, NODES_PER_TILE)])
        plsc.subcore_barrier()

        @pl.loop(g_lo, g_hi)
        def _group(g):
            eb = g * CHUNK
            pltpu.sync_copy(eg_hbm.at[pl.ds(eb, CHUNK)], srcb)
            pltpu.sync_copy(eg_hbm.at[pl.ds(E_EDGES + eb, CHUNK)], dstb)
            for b in range(CHUNK // 16):
                sl = pl.ds(16 * b, 16)
                vidxb[sl] = srcb[sl] * 2 + c
            pltpu.sync_copy(v_hbm.at[vidxb], vbuf)
            pltpu.sync_copy(e_hbm.at[pl.ds(eb, CHUNK)], ebuf)
            for i in range(CHUNK):
                iv = jnp.full((16,), i, jnp.int32)
                for j2 in range(4):
                    hv = jnp.full((16,), base_h + j2, jnp.int32)
                    sv = plsc.load_gather(ebuf, [iv, hv])
                    for half in range(2):
                        off = 32 * j2 + 16 * half
                        mbuf[i, pl.ds(off, 16)] = (
                            vbuf[i, pl.ds(off, 16)] * sv)
            pltpu.sync_copy(mbuf, num_sh.at[dstb], add=True)

        plsc.subcore_barrier()

        @pl.loop(0, NODES_PER_TILE // CHUNK)
        def _drain(t):
            nb = nbase + CHUNK * t
            pltpu.sync_copy(num_sh.at[pl.ds(nb, CHUNK)], nbuf)
            pltpu.sync_copy(den_hbm.at[0, ridx, pl.ds(nb, CHUNK)], d0b)
            pltpu.sync_copy(den_hbm.at[1, ridx, pl.ds(nb, CHUNK)], d1b)

            @pl.loop(0, CHUNK)
            def _node(n):
                nv = jnp.full((16,), n, jnp.int32)
                for j2 in range(4):
                    hv = jnp.full((16,), base_h + j2, jnp.int32)
                    sden = (plsc.load_gather(d0b, [nv, hv])
                            + plsc.load_gather(d1b, [nv, hv]) + 1e-9)
                    rv = 1.0 / sden
                    for half in range(2):
                        off = 32 * j2 + 16 * half
                        nbuf[n, pl.ds(off, 16)] = nbuf[n, pl.ds(off, 16)] * rv
            pltpu.sync_copy(nbuf, agg_hbm.at[c, pl.ds(nb, CHUNK)])
        plsc.subcore_barrier()


def kernel(h_paper, h_author, edge_writes, edge_cites, Wk, bk, Wv, bv, Wq, bq,
           Wa, ba, rel_att, rel_msg, rel_pri, skip):
    sqrt_dk = math.sqrt(D_K)
    att0 = rel_att[0] * (rel_pri[0][:, None, None] / sqrt_dk)
    att1 = rel_att[1] * (rel_pri[1][:, None, None] / sqrt_dk)
    Wk0, bk0 = _fold(Wk[1], bk[1], att0)
    Wv0, bv0 = _fold(Wv[1], bv[1], rel_msg[0])
    Wk1, bk1 = _fold(Wk[0], bk[0], att1)
    Wv1, bv1 = _fold(Wv[0], bv[0], rel_msg[1])

    K0 = _mm_bias(h_author, Wk0, bk0)
    V0 = _mm_bias(h_author, Wv0, bv0)
    K1 = _mm_bias(h_paper, Wk1, bk1)
    V1 = _mm_bias(h_paper, Wv1, bv1)
    Q = _mm_bias(h_paper, Wq[0], bq[0])

    z8 = jnp.zeros((NPAD, N_HEADS), jnp.float32)
    ew_flat = edge_writes.reshape(-1)
    ec_flat = edge_cites.reshape(-1)
    e0, e1, den = _scores_sc(K0, Q, K1, ew_flat, ec_flat, z8)

    zbig = jnp.zeros((NPAD, HALF), jnp.float32)
    V0f = V0.reshape(N_AUTHOR, 2, HALF).reshape(2 * N_AUTHOR, HALF)
    V1f = V1.reshape(N_PAPER, 2, HALF).reshape(2 * N_PAPER, HALF)
    agg0h, agg1h = _messages_sc(ew_flat, ec_flat, e0, e1, V0f, V1f, den, zbig)
    agg0 = jnp.concatenate([agg0h[0, :N_PAPER], agg0h[1, :N_PAPER]], axis=1)
    agg1 = jnp.concatenate([agg1h[0, :N_PAPER], agg1h[1, :N_PAPER]], axis=1)

    out_p, out_a = _final(agg0, agg1, h_paper, h_author, 0.5 * Wa[0],
                          ba[0], ba[1], skip)
    return (out_p, out_a)


# speedup vs baseline: 9.1241x; 1.1515x over previous
import math
import functools
import jax
import jax.numpy as jnp
from jax import lax
from jax.experimental import pallas as pl
from jax.experimental.pallas import tpu as pltpu
from jax.experimental.pallas import tpu_sc as plsc

N_PAPER = 10000
N_AUTHOR = 10000
E_EDGES = 160000
IN_DIM = 256
OUT_DIM = 256
N_HEADS = 8
D_K = OUT_DIM // N_HEADS
NTILE = 32
CHUNK = 128
NGROUP = E_EDGES // CHUNK
LOG2_NTILE = 5
LOG2_NSUB = 4
NPAD = 10240
NODES_PER_TILE = NPAD // 16


def _mm_bias_kernel(x_ref, w_ref, b_ref, o_ref):
    o_ref[...] = jnp.dot(x_ref[...], w_ref[...],
                         preferred_element_type=jnp.float32) + b_ref[...]


def _mm_bias(x, w, b, block_rows=2000):
    n = x.shape[0]
    return pl.pallas_call(
        _mm_bias_kernel,
        grid=(n // block_rows,),
        in_specs=[
            pl.BlockSpec((block_rows, x.shape[1]), lambda i: (i, 0)),
            pl.BlockSpec((w.shape[0], w.shape[1]), lambda i: (0, 0)),
            pl.BlockSpec((1, w.shape[1]), lambda i: (0, 0)),
        ],
        out_specs=pl.BlockSpec((block_rows, w.shape[1]), lambda i: (i, 0)),
        out_shape=jax.ShapeDtypeStruct((n, w.shape[1]), jnp.float32),
    )(x, w, b.reshape(1, -1))


def _final_kernel(a0_ref, a1_ref, hp_ref, ha_ref, wa0_ref, ba0_ref, ba1_ref,
                  sk_ref, op_ref, oa_ref):
    alpha0 = sk_ref[0, 0]
    alpha1 = sk_ref[0, 1]
    agg = a0_ref[...] + a1_ref[...]
    t = jnp.dot(agg, wa0_ref[...],
                preferred_element_type=jnp.float32) + ba0_ref[...]
    op_ref[...] = t * alpha0 + hp_ref[...] * (1.0 - alpha0)
    oa_ref[...] = ba1_ref[...] * alpha1 + ha_ref[...] * (1.0 - alpha1)


def _final(agg0, agg1, h_paper, h_author, Wa0h, ba0, ba1, skip,
           block_rows=2000):
    n = N_PAPER
    alphas = jax.nn.sigmoid(skip).reshape(1, 2)
    return pl.pallas_call(
        _final_kernel,
        grid=(n // block_rows,),
        in_specs=[
            pl.BlockSpec((block_rows, OUT_DIM), lambda i: (i, 0)),
            pl.BlockSpec((block_rows, OUT_DIM), lambda i: (i, 0)),
            pl.BlockSpec((block_rows, IN_DIM), lambda i: (i, 0)),
            pl.BlockSpec((block_rows, IN_DIM), lambda i: (i, 0)),
            pl.BlockSpec((OUT_DIM, OUT_DIM), lambda i: (0, 0)),
            pl.BlockSpec((1, OUT_DIM), lambda i: (0, 0)),
            pl.BlockSpec((1, OUT_DIM), lambda i: (0, 0)),
            pl.BlockSpec((1, 2), lambda i: (0, 0)),
        ],
        out_specs=[
            pl.BlockSpec((block_rows, OUT_DIM), lambda i: (i, 0)),
            pl.BlockSpec((block_rows, OUT_DIM), lambda i: (i, 0)),
        ],
        out_shape=[
            jax.ShapeDtypeStruct((n, OUT_DIM), jnp.float32),
            jax.ShapeDtypeStruct((n, OUT_DIM), jnp.float32),
        ],
    )(agg0, agg1, h_paper, h_author, Wa0h, ba0.reshape(1, -1),
      ba1.reshape(1, -1), alphas)


def _fold(W, b, rel):
    Wf = jnp.einsum('ihj,hjk->ihk', W.reshape(IN_DIM, N_HEADS, D_K),
                    rel).reshape(IN_DIM, OUT_DIM)
    bf = jnp.einsum('hj,hjk->hk', b.reshape(N_HEADS, D_K), rel).reshape(OUT_DIM)
    return Wf, bf


_SC_MESH = plsc.VectorSubcoreMesh(core_axis_name="c", subcore_axis_name="s")


@functools.partial(
    pl.kernel,
    out_type=[
        jax.ShapeDtypeStruct((E_EDGES, N_HEADS), jnp.float32),
        jax.ShapeDtypeStruct((E_EDGES, N_HEADS), jnp.float32),
        jax.ShapeDtypeStruct((2, 2, NPAD, N_HEADS), jnp.float32),
    ],
    mesh=_SC_MESH,
    compiler_params=pltpu.CompilerParams(use_tc_tiling_on_sc=False,
                                         needs_layout_passes=False),
    scratch_types=[
        pltpu.VMEM((CHUNK,), jnp.int32),
        pltpu.VMEM((CHUNK,), jnp.int32),
        pltpu.VMEM((CHUNK, IN_DIM), jnp.float32),
        pltpu.VMEM((CHUNK, IN_DIM), jnp.float32),
        pltpu.VMEM((CHUNK, N_HEADS), jnp.float32),
        pltpu.VMEM_SHARED((NPAD, N_HEADS), jnp.float32),
    ],
)
def _scores_sc(k0_hbm, q_hbm, k1_hbm, eg0_hbm, eg1_hbm, z8_hbm,
               e0_hbm, e1_hbm, den_hbm,
               srcb, dstb, kbuf, qbuf, ebuf, den_sh):
    c = lax.axis_index("c")
    s = lax.axis_index("s")
    w = s * 2 + c
    lane16 = lax.iota(jnp.int32, 16)
    g_lo = lax.shift_right_logical(NGROUP * w, LOG2_NTILE)
    g_hi = lax.shift_right_logical(NGROUP * (w + 1), LOG2_NTILE)
    nbase = NODES_PER_TILE * s

    for ridx, (eg_hbm, k_hbm, e_hbm) in enumerate(
            [(eg0_hbm, k0_hbm, e0_hbm), (eg1_hbm, k1_hbm, e1_hbm)]):
        pltpu.sync_copy(z8_hbm.at[pl.ds(nbase, NODES_PER_TILE)],
                        den_sh.at[pl.ds(nbase, NODES_PER_TILE)])
        plsc.subcore_barrier()

        @pl.loop(g_lo, g_hi)
        def _group(g):
            eb = g * CHUNK
            pltpu.sync_copy(eg_hbm.at[pl.ds(eb, CHUNK)], srcb)
            pltpu.sync_copy(eg_hbm.at[pl.ds(E_EDGES + eb, CHUNK)], dstb)
            pltpu.sync_copy(k_hbm.at[srcb], kbuf)
            pltpu.sync_copy(q_hbm.at[dstb], qbuf)
            @pl.loop(0, CHUNK // 16)
            def _sub(b):
                lanes = lane16 + 16 * b
                for h in range(N_HEADS):
                    acc = jnp.zeros((16,), jnp.float32)
                    for d2 in range(D_K):
                        d = jnp.full((16,), D_K * h + d2, jnp.int32)
                        qc = plsc.load_gather(qbuf, [lanes, d])
                        kc = plsc.load_gather(kbuf, [lanes, d])
                        acc = acc + qc * kc
                    eh = jnp.exp(acc)
                    plsc.store_scatter(
                        ebuf, [lanes, jnp.full((16,), h, jnp.int32)], eh)
            pltpu.sync_copy(ebuf, e_hbm.at[pl.ds(eb, CHUNK)])
            pltpu.sync_copy(ebuf, den_sh.at[dstb], add=True)

        plsc.subcore_barrier()
        pltpu.sync_copy(den_sh.at[pl.ds(nbase, NODES_PER_TILE)],
                        den_hbm.at[c, ridx, pl.ds(nbase, NODES_PER_TILE)])
        plsc.subcore_barrier()


HALF = 128


@functools.partial(
    pl.kernel,
    out_type=[
        jax.ShapeDtypeStruct((2, NPAD, HALF), jnp.float32),
        jax.ShapeDtypeStruct((2, NPAD, HALF), jnp.float32),
    ],
    mesh=_SC_MESH,
    compiler_params=pltpu.CompilerParams(use_tc_tiling_on_sc=False,
                                         needs_layout_passes=False),
    scratch_types=[
        pltpu.VMEM((CHUNK,), jnp.int32),
        pltpu.VMEM((CHUNK,), jnp.int32),
        pltpu.VMEM((CHUNK,), jnp.int32),
        pltpu.VMEM((CHUNK, HALF), jnp.float32),
        pltpu.VMEM((CHUNK, N_HEADS), jnp.float32),
        pltpu.VMEM((CHUNK, N_HEADS), jnp.float32),
        pltpu.VMEM((CHUNK, N_HEADS), jnp.float32),
        pltpu.VMEM_SHARED((NPAD, HALF), jnp.float32),
    ],
)
def _messages_sc(eg0_hbm, eg1_hbm, e0_hbm, e1_hbm, v0_hbm, v1_hbm, den_hbm,
                 zbig_hbm, agg0_hbm, agg1_hbm,
                 srcb, dstb, vidxb, vbuf, ebuf, d0b, d1b, num_sh):
    c = lax.axis_index("c")
    s = lax.axis_index("s")
    g_lo = lax.shift_right_logical(NGROUP * s, LOG2_NSUB)
    g_hi = lax.shift_right_logical(NGROUP * (s + 1), LOG2_NSUB)
    nbase = NODES_PER_TILE * s
    base_h = 4 * c

    for ridx, (eg_hbm, e_hbm, v_hbm, agg_hbm) in enumerate(
            [(eg0_hbm, e0_hbm, v0_hbm, agg0_hbm),
             (eg1_hbm, e1_hbm, v1_hbm, agg1_hbm)]):
        pltpu.sync_copy(zbig_hbm.at[pl.ds(nbase, NODES_PER_TILE)],
                        num_sh.at[pl.ds(nbase, NODES_PER_TILE)])
        plsc.subcore_barrier()

        @pl.loop(g_lo, g_hi)
        def _group(g):
            eb = g * CHUNK
            pltpu.sync_copy(eg_hbm.at[pl.ds(eb, CHUNK)], srcb)
            pltpu.sync_copy(eg_hbm.at[pl.ds(E_EDGES + eb, CHUNK)], dstb)
            for b in range(CHUNK // 16):
                sl = pl.ds(16 * b, 16)
                vidxb[sl] = srcb[sl] * 2 + c
            pltpu.sync_copy(v_hbm.at[vidxb], vbuf)
            pltpu.sync_copy(e_hbm.at[pl.ds(eb, CHUNK)], ebuf)

            @pl.loop(0, CHUNK)
            def _edge(i):
                iv = jnp.full((16,), i, jnp.int32)
                for j2 in range(4):
                    hv = jnp.full((16,), base_h + j2, jnp.int32)
                    sv = plsc.load_gather(ebuf, [iv, hv])
                    for half in range(2):
                        off = 32 * j2 + 16 * half
                        vbuf[i, pl.ds(off, 16)] = (
                            vbuf[i, pl.ds(off, 16)] * sv)
            pltpu.sync_copy(vbuf, num_sh.at[dstb], add=True)

        plsc.subcore_barrier()

        @pl.loop(0, NODES_PER_TILE // CHUNK)
        def _drain(t):
            nb = nbase + CHUNK * t
            pltpu.sync_copy(num_sh.at[pl.ds(nb, CHUNK)], vbuf)
            pltpu.sync_copy(den_hbm.at[0, ridx, pl.ds(nb, CHUNK)], d0b)
            pltpu.sync_copy(den_hbm.at[1, ridx, pl.ds(nb, CHUNK)], d1b)

            @pl.loop(0, CHUNK)
            def _node(n):
                nv = jnp.full((16,), n, jnp.int32)
                for j2 in range(4):
                    hv = jnp.full((16,), base_h + j2, jnp.int32)
                    sden = (plsc.load_gather(d0b, [nv, hv])
                            + plsc.load_gather(d1b, [nv, hv]) + 1e-9)
                    rv = 1.0 / sden
                    for half in range(2):
                        off = 32 * j2 + 16 * half
                        vbuf[n, pl.ds(off, 16)] = vbuf[n, pl.ds(off, 16)] * rv
            pltpu.sync_copy(vbuf, agg_hbm.at[c, pl.ds(nb, CHUNK)])
        plsc.subcore_barrier()


def kernel(h_paper, h_author, edge_writes, edge_cites, Wk, bk, Wv, bv, Wq, bq,
           Wa, ba, rel_att, rel_msg, rel_pri, skip):
    sqrt_dk = math.sqrt(D_K)
    att0 = rel_att[0] * (rel_pri[0][:, None, None] / sqrt_dk)
    att1 = rel_att[1] * (rel_pri[1][:, None, None] / sqrt_dk)
    Wk0, bk0 = _fold(Wk[1], bk[1], att0)
    Wv0, bv0 = _fold(Wv[1], bv[1], rel_msg[0])
    Wk1, bk1 = _fold(Wk[0], bk[0], att1)
    Wv1, bv1 = _fold(Wv[0], bv[0], rel_msg[1])

    K0 = _mm_bias(h_author, Wk0, bk0)
    V0 = _mm_bias(h_author, Wv0, bv0)
    K1 = _mm_bias(h_paper, Wk1, bk1)
    V1 = _mm_bias(h_paper, Wv1, bv1)
    Q = _mm_bias(h_paper, Wq[0], bq[0])

    z8 = jnp.zeros((NPAD, N_HEADS), jnp.float32)
    ew_flat = edge_writes.reshape(-1)
    ec_flat = edge_cites.reshape(-1)
    e0, e1, den = _scores_sc(K0, Q, K1, ew_flat, ec_flat, z8)

    zbig = jnp.zeros((NPAD, HALF), jnp.float32)
    V0f = V0.reshape(N_AUTHOR, 2, HALF).reshape(2 * N_AUTHOR, HALF)
    V1f = V1.reshape(N_PAPER, 2, HALF).reshape(2 * N_PAPER, HALF)
    agg0h, agg1h = _messages_sc(ew_flat, ec_flat, e0, e1, V0f, V1f, den, zbig)
    agg0 = jnp.concatenate([agg0h[0, :N_PAPER], agg0h[1, :N_PAPER]], axis=1)
    agg1 = jnp.concatenate([agg1h[0, :N_PAPER], agg1h[1, :N_PAPER]], axis=1)

    out_p, out_a = _final(agg0, agg1, h_paper, h_author, 0.5 * Wa[0],
                          ba[0], ba[1], skip)
    return (out_p, out_a)


# speedup vs baseline: 9.9538x; 1.0909x over previous
import math
import functools
import jax
import jax.numpy as jnp
from jax import lax
from jax.experimental import pallas as pl
from jax.experimental.pallas import tpu as pltpu
from jax.experimental.pallas import tpu_sc as plsc

N_PAPER = 10000
N_AUTHOR = 10000
E_EDGES = 160000
IN_DIM = 256
OUT_DIM = 256
N_HEADS = 8
D_K = OUT_DIM // N_HEADS
NTILE = 32
CHUNK = 128
NGROUP = E_EDGES // CHUNK
LOG2_NTILE = 5
LOG2_NSUB = 4
NPAD = 10240
NODES_PER_TILE = NPAD // 16


def _mm_bias_kernel(x_ref, w_ref, b_ref, o_ref):
    o_ref[...] = jnp.dot(x_ref[...], w_ref[...],
                         preferred_element_type=jnp.float32) + b_ref[...]


def _mm_bias(x, w, b, block_rows=2000):
    n = x.shape[0]
    return pl.pallas_call(
        _mm_bias_kernel,
        grid=(n // block_rows,),
        in_specs=[
            pl.BlockSpec((block_rows, x.shape[1]), lambda i: (i, 0)),
            pl.BlockSpec((w.shape[0], w.shape[1]), lambda i: (0, 0)),
            pl.BlockSpec((1, w.shape[1]), lambda i: (0, 0)),
        ],
        out_specs=pl.BlockSpec((block_rows, w.shape[1]), lambda i: (i, 0)),
        out_shape=jax.ShapeDtypeStruct((n, w.shape[1]), jnp.float32),
    )(x, w, b.reshape(1, -1))


def _final_kernel(a0_ref, a1_ref, hp_ref, ha_ref, wa0_ref, ba0_ref, ba1_ref,
                  sk_ref, op_ref, oa_ref):
    alpha0 = sk_ref[0, 0]
    alpha1 = sk_ref[0, 1]
    agg = a0_ref[...] + a1_ref[...]
    t = jnp.dot(agg, wa0_ref[...],
                preferred_element_type=jnp.float32) + ba0_ref[...]
    op_ref[...] = t * alpha0 + hp_ref[...] * (1.0 - alpha0)
    oa_ref[...] = ba1_ref[...] * alpha1 + ha_ref[...] * (1.0 - alpha1)


def _final(agg0, agg1, h_paper, h_author, Wa0h, ba0, ba1, skip,
           block_rows=2000):
    n = N_PAPER
    alphas = jax.nn.sigmoid(skip).reshape(1, 2)
    return pl.pallas_call(
        _final_kernel,
        grid=(n // block_rows,),
        in_specs=[
            pl.BlockSpec((block_rows, OUT_DIM), lambda i: (i, 0)),
            pl.BlockSpec((block_rows, OUT_DIM), lambda i: (i, 0)),
            pl.BlockSpec((block_rows, IN_DIM), lambda i: (i, 0)),
            pl.BlockSpec((block_rows, IN_DIM), lambda i: (i, 0)),
            pl.BlockSpec((OUT_DIM, OUT_DIM), lambda i: (0, 0)),
            pl.BlockSpec((1, OUT_DIM), lambda i: (0, 0)),
            pl.BlockSpec((1, OUT_DIM), lambda i: (0, 0)),
            pl.BlockSpec((1, 2), lambda i: (0, 0)),
        ],
        out_specs=[
            pl.BlockSpec((block_rows, OUT_DIM), lambda i: (i, 0)),
            pl.BlockSpec((block_rows, OUT_DIM), lambda i: (i, 0)),
        ],
        out_shape=[
            jax.ShapeDtypeStruct((n, OUT_DIM), jnp.float32),
            jax.ShapeDtypeStruct((n, OUT_DIM), jnp.float32),
        ],
    )(agg0, agg1, h_paper, h_author, Wa0h, ba0.reshape(1, -1),
      ba1.reshape(1, -1), alphas)


def _fold(W, b, rel):
    Wf = jnp.einsum('ihj,hjk->ihk', W.reshape(IN_DIM, N_HEADS, D_K),
                    rel).reshape(IN_DIM, OUT_DIM)
    bf = jnp.einsum('hj,hjk->hk', b.reshape(N_HEADS, D_K), rel).reshape(OUT_DIM)
    return Wf, bf


_SC_MESH = plsc.VectorSubcoreMesh(core_axis_name="c", subcore_axis_name="s")


@functools.partial(
    pl.kernel,
    out_type=[
        jax.ShapeDtypeStruct((E_EDGES, N_HEADS), jnp.float32),
        jax.ShapeDtypeStruct((E_EDGES, N_HEADS), jnp.float32),
        jax.ShapeDtypeStruct((2, 2, NPAD, N_HEADS), jnp.float32),
    ],
    mesh=_SC_MESH,
    compiler_params=pltpu.CompilerParams(use_tc_tiling_on_sc=False,
                                         needs_layout_passes=False),
    scratch_types=[
        pltpu.VMEM((CHUNK,), jnp.int32),
        pltpu.VMEM((CHUNK,), jnp.int32),
        pltpu.VMEM((CHUNK, IN_DIM), jnp.float32),
        pltpu.VMEM((CHUNK, IN_DIM), jnp.float32),
        pltpu.VMEM((CHUNK, N_HEADS), jnp.float32),
        pltpu.VMEM_SHARED((NPAD, N_HEADS), jnp.float32),
    ],
)
def _scores_sc(k0_hbm, q_hbm, k1_hbm, eg0_hbm, eg1_hbm, z8_hbm,
               e0_hbm, e1_hbm, den_hbm,
               srcb, dstb, kbuf, qbuf, ebuf, den_sh):
    c = lax.axis_index("c")
    s = lax.axis_index("s")
    w = s * 2 + c
    lane16 = lax.iota(jnp.int32, 16)
    g_lo = lax.shift_right_logical(NGROUP * w, LOG2_NTILE)
    g_hi = lax.shift_right_logical(NGROUP * (w + 1), LOG2_NTILE)
    nbase = NODES_PER_TILE * s

    for ridx, (eg_hbm, k_hbm, e_hbm) in enumerate(
            [(eg0_hbm, k0_hbm, e0_hbm), (eg1_hbm, k1_hbm, e1_hbm)]):
        pltpu.sync_copy(z8_hbm.at[pl.ds(nbase, NODES_PER_TILE)],
                        den_sh.at[pl.ds(nbase, NODES_PER_TILE)])
        plsc.subcore_barrier()

        @pl.loop(g_lo, g_hi)
        def _group(g):
            eb = g * CHUNK
            pltpu.sync_copy(eg_hbm.at[pl.ds(eb, CHUNK)], srcb)
            pltpu.sync_copy(eg_hbm.at[pl.ds(E_EDGES + eb, CHUNK)], dstb)
            pltpu.sync_copy(k_hbm.at[srcb], kbuf)
            pltpu.sync_copy(q_hbm.at[dstb], qbuf)
            @pl.loop(0, CHUNK // 16)
            def _sub(b):
                lanes = lane16 + 16 * b
                for h in range(N_HEADS):
                    acc = jnp.zeros((16,), jnp.float32)
                    for d2 in range(D_K):
                        d = jnp.full((16,), D_K * h + d2, jnp.int32)
                        qc = plsc.load_gather(qbuf, [lanes, d])
                        kc = plsc.load_gather(kbuf, [lanes, d])
                        acc = acc + qc * kc
                    eh = jnp.exp(acc)
                    plsc.store_scatter(
                        ebuf, [lanes, jnp.full((16,), h, jnp.int32)], eh)
            pltpu.sync_copy(ebuf, e_hbm.at[pl.ds(eb, CHUNK)])
            pltpu.sync_copy(ebuf, den_sh.at[dstb], add=True)

        plsc.subcore_barrier()
        pltpu.sync_copy(den_sh.at[pl.ds(nbase, NODES_PER_TILE)],
                        den_hbm.at[c, ridx, pl.ds(nbase, NODES_PER_TILE)])
        plsc.subcore_barrier()


HALF = 128


@functools.partial(
    pl.kernel,
    out_type=[
        jax.ShapeDtypeStruct((2, NPAD, HALF), jnp.float32),
        jax.ShapeDtypeStruct((2, NPAD, HALF), jnp.float32),
    ],
    mesh=_SC_MESH,
    compiler_params=pltpu.CompilerParams(use_tc_tiling_on_sc=False,
                                         needs_layout_passes=False),
    scratch_types=[
        pltpu.VMEM((CHUNK,), jnp.int32),
        pltpu.VMEM((CHUNK,), jnp.int32),
        pltpu.VMEM((CHUNK,), jnp.int32),
        pltpu.VMEM((CHUNK,), jnp.int32),
        pltpu.VMEM((CHUNK,), jnp.int32),
        pltpu.VMEM((CHUNK,), jnp.int32),
        pltpu.VMEM((CHUNK, HALF), jnp.float32),
        pltpu.VMEM((CHUNK, HALF), jnp.float32),
        pltpu.VMEM((CHUNK, N_HEADS), jnp.float32),
        pltpu.VMEM((CHUNK, N_HEADS), jnp.float32),
        pltpu.VMEM((CHUNK, N_HEADS), jnp.float32),
        pltpu.VMEM((CHUNK, N_HEADS), jnp.float32),
        pltpu.VMEM_SHARED((NPAD, HALF), jnp.float32),
        pltpu.SemaphoreType.DMA,
        pltpu.SemaphoreType.DMA,
        pltpu.SemaphoreType.DMA,
        pltpu.SemaphoreType.DMA,
    ],
)
def _messages_sc(eg0_hbm, eg1_hbm, e0_hbm, e1_hbm, v0_hbm, v1_hbm, den_hbm,
                 zbig_hbm, agg0_hbm, agg1_hbm,
                 srcb0, srcb1, dstb0, dstb1, vidx0, vidx1, vbuf0, vbuf1,
                 ebuf0, ebuf1, d0b, d1b, num_sh,
                 sidx0, sidx1, sdat0, sdat1):
    c = lax.axis_index("c")
    s = lax.axis_index("s")
    g_lo = lax.shift_right_logical(NGROUP * s, LOG2_NSUB)
    g_hi = lax.shift_right_logical(NGROUP * (s + 1), LOG2_NSUB)
    nbase = NODES_PER_TILE * s
    base_h = 4 * c
    srcb = [srcb0, srcb1]
    dstb = [dstb0, dstb1]
    vidxb = [vidx0, vidx1]
    vbuf = [vbuf0, vbuf1]
    ebuf = [ebuf0, ebuf1]
    sidx = [sidx0, sidx1]
    sdat = [sdat0, sdat1]

    for ridx, (eg_hbm, e_hbm, v_hbm, agg_hbm) in enumerate(
            [(eg0_hbm, e0_hbm, v0_hbm, agg0_hbm),
             (eg1_hbm, e1_hbm, v1_hbm, agg1_hbm)]):
        pltpu.sync_copy(zbig_hbm.at[pl.ds(nbase, NODES_PER_TILE)],
                        num_sh.at[pl.ds(nbase, NODES_PER_TILE)])
        plsc.subcore_barrier()

        def issue_idx(g, p):
            eb = g * CHUNK
            pltpu.async_copy(eg_hbm.at[pl.ds(eb, CHUNK)], srcb[p], sidx[p])
            pltpu.async_copy(eg_hbm.at[pl.ds(E_EDGES + eb, CHUNK)],
                             dstb[p], sidx[p])

        def wait_idx(p):
            pltpu.make_async_copy(eg_hbm.at[pl.ds(0, CHUNK)],
                                  srcb[p], sidx[p]).wait()
            pltpu.make_async_copy(eg_hbm.at[pl.ds(0, CHUNK)],
                                  dstb[p], sidx[p]).wait()

        def compute_vidx(p):
            for b in range(CHUNK // 16):
                sl = pl.ds(16 * b, 16)
                vidxb[p][sl] = srcb[p][sl] * 2 + c

        def issue_data(g, p):
            eb = g * CHUNK
            pltpu.async_copy(v_hbm.at[vidxb[p]], vbuf[p], sdat[p])
            pltpu.async_copy(e_hbm.at[pl.ds(eb, CHUNK)], ebuf[p], sdat[p])

        def wait_data(p):
            pltpu.make_async_copy(v_hbm.at[vidxb[p]], vbuf[p], sdat[p]).wait()
            pltpu.make_async_copy(e_hbm.at[pl.ds(0, CHUNK)],
                                  ebuf[p], sdat[p]).wait()

        def step(g, p):
            @pl.when(g < g_hi)
            def _():
                wait_data(p)

                @pl.when(g + 1 < g_hi)
                def _():
                    wait_idx(1 - p)
                    compute_vidx(1 - p)
                    issue_data(g + 1, 1 - p)

                @pl.loop(0, CHUNK)
                def _edge(i):
                    iv = jnp.full((16,), i, jnp.int32)
                    for j2 in range(4):
                        hv = jnp.full((16,), base_h + j2, jnp.int32)
                        sv = plsc.load_gather(ebuf[p], [iv, hv])
                        for half in range(2):
                            off = 32 * j2 + 16 * half
                            vbuf[p][i, pl.ds(off, 16)] = (
                                vbuf[p][i, pl.ds(off, 16)] * sv)
                pltpu.sync_copy(vbuf[p], num_sh.at[dstb[p]], add=True)

                @pl.when(g + 2 < g_hi)
                def _():
                    issue_idx(g + 2, p)

        pltpu.sync_copy(eg_hbm.at[pl.ds(g_lo * CHUNK, CHUNK)], srcb[0])
        pltpu.sync_copy(eg_hbm.at[pl.ds(E_EDGES + g_lo * CHUNK, CHUNK)],
                        dstb[0])
        compute_vidx(0)
        issue_data(g_lo, 0)

        @pl.when(g_lo + 1 < g_hi)
        def _():
            issue_idx(g_lo + 1, 1)

        npair = lax.shift_right_logical(g_hi - g_lo + 1, 1)

        @pl.loop(0, npair)
        def _pair(t):
            g0 = g_lo + 2 * t
            step(g0, 0)
            step(g0 + 1, 1)

        plsc.subcore_barrier()

        @pl.loop(0, NODES_PER_TILE // CHUNK)
        def _drain(t):
            nb = nbase + CHUNK * t
            pltpu.sync_copy(num_sh.at[pl.ds(nb, CHUNK)], vbuf0)
            pltpu.sync_copy(den_hbm.at[0, ridx, pl.ds(nb, CHUNK)], d0b)
            pltpu.sync_copy(den_hbm.at[1, ridx, pl.ds(nb, CHUNK)], d1b)

            @pl.loop(0, CHUNK)
            def _node(n):
                nv = jnp.full((16,), n, jnp.int32)
                for j2 in range(4):
                    hv = jnp.full((16,), base_h + j2, jnp.int32)
                    sden = (plsc.load_gather(d0b, [nv, hv])
                            + plsc.load_gather(d1b, [nv, hv]) + 1e-9)
                    rv = 1.0 / sden
                    for half in range(2):
                        off = 32 * j2 + 16 * half
                        vbuf0[n, pl.ds(off, 16)] = vbuf0[n, pl.ds(off, 16)] * rv
            pltpu.sync_copy(vbuf0, agg_hbm.at[c, pl.ds(nb, CHUNK)])
        plsc.subcore_barrier()


def kernel(h_paper, h_author, edge_writes, edge_cites, Wk, bk, Wv, bv, Wq, bq,
           Wa, ba, rel_att, rel_msg, rel_pri, skip):
    sqrt_dk = math.sqrt(D_K)
    att0 = rel_att[0] * (rel_pri[0][:, None, None] / sqrt_dk)
    att1 = rel_att[1] * (rel_pri[1][:, None, None] / sqrt_dk)
    Wk0, bk0 = _fold(Wk[1], bk[1], att0)
    Wv0, bv0 = _fold(Wv[1], bv[1], rel_msg[0])
    Wk1, bk1 = _fold(Wk[0], bk[0], att1)
    Wv1, bv1 = _fold(Wv[0], bv[0], rel_msg[1])

    K0 = _mm_bias(h_author, Wk0, bk0)
    V0 = _mm_bias(h_author, Wv0, bv0)
    K1 = _mm_bias(h_paper, Wk1, bk1)
    V1 = _mm_bias(h_paper, Wv1, bv1)
    Q = _mm_bias(h_paper, Wq[0], bq[0])

    z8 = jnp.zeros((NPAD, N_HEADS), jnp.float32)
    ew_flat = edge_writes.reshape(-1)
    ec_flat = edge_cites.reshape(-1)
    e0, e1, den = _scores_sc(K0, Q, K1, ew_flat, ec_flat, z8)

    zbig = jnp.zeros((NPAD, HALF), jnp.float32)
    V0f = V0.reshape(N_AUTHOR, 2, HALF).reshape(2 * N_AUTHOR, HALF)
    V1f = V1.reshape(N_PAPER, 2, HALF).reshape(2 * N_PAPER, HALF)
    agg0h, agg1h = _messages_sc(ew_flat, ec_flat, e0, e1, V0f, V1f, den, zbig)
    agg0 = jnp.concatenate([agg0h[0, :N_PAPER], agg0h[1, :N_PAPER]], axis=1)
    agg1 = jnp.concatenate([agg1h[0, :N_PAPER], agg1h[1, :N_PAPER]], axis=1)

    out_p, out_a = _final(agg0, agg1, h_paper, h_author, 0.5 * Wa[0],
                          ba[0], ba[1], skip)
    return (out_p, out_a)


# speedup vs baseline: 10.9048x; 1.0955x over previous
import math
import functools
import jax
import jax.numpy as jnp
from jax import lax
from jax.experimental import pallas as pl
from jax.experimental.pallas import tpu as pltpu
from jax.experimental.pallas import tpu_sc as plsc

N_PAPER = 10000
N_AUTHOR = 10000
E_EDGES = 160000
IN_DIM = 256
OUT_DIM = 256
N_HEADS = 8
D_K = OUT_DIM // N_HEADS
NTILE = 32
CHUNK = 128
NGROUP = E_EDGES // CHUNK
ACHUNK = 80
ANGROUP = E_EDGES // ACHUNK
LOG2_NTILE = 5
LOG2_NSUB = 4
NPAD = 10240
NODES_PER_TILE = NPAD // 16


def _mm_bias_kernel(x_ref, w_ref, b_ref, o_ref):
    o_ref[...] = jnp.dot(x_ref[...], w_ref[...],
                         preferred_element_type=jnp.float32) + b_ref[...]


def _mm_bias(x, w, b, block_rows=2000):
    n = x.shape[0]
    return pl.pallas_call(
        _mm_bias_kernel,
        grid=(n // block_rows,),
        in_specs=[
            pl.BlockSpec((block_rows, x.shape[1]), lambda i: (i, 0)),
            pl.BlockSpec((w.shape[0], w.shape[1]), lambda i: (0, 0)),
            pl.BlockSpec((1, w.shape[1]), lambda i: (0, 0)),
        ],
        out_specs=pl.BlockSpec((block_rows, w.shape[1]), lambda i: (i, 0)),
        out_shape=jax.ShapeDtypeStruct((n, w.shape[1]), jnp.float32),
    )(x, w, b.reshape(1, -1))


def _final_kernel(a0_ref, a1_ref, hp_ref, ha_ref, wa0_ref, ba0_ref, ba1_ref,
                  sk_ref, op_ref, oa_ref):
    alpha0 = sk_ref[0, 0]
    alpha1 = sk_ref[0, 1]
    agg = a0_ref[...] + a1_ref[...]
    t = jnp.dot(agg, wa0_ref[...],
                preferred_element_type=jnp.float32) + ba0_ref[...]
    op_ref[...] = t * alpha0 + hp_ref[...] * (1.0 - alpha0)
    oa_ref[...] = ba1_ref[...] * alpha1 + ha_ref[...] * (1.0 - alpha1)


def _final(agg0, agg1, h_paper, h_author, Wa0h, ba0, ba1, skip,
           block_rows=2000):
    n = N_PAPER
    alphas = jax.nn.sigmoid(skip).reshape(1, 2)
    return pl.pallas_call(
        _final_kernel,
        grid=(n // block_rows,),
        in_specs=[
            pl.BlockSpec((block_rows, OUT_DIM), lambda i: (i, 0)),
            pl.BlockSpec((block_rows, OUT_DIM), lambda i: (i, 0)),
            pl.BlockSpec((block_rows, IN_DIM), lambda i: (i, 0)),
            pl.BlockSpec((block_rows, IN_DIM), lambda i: (i, 0)),
            pl.BlockSpec((OUT_DIM, OUT_DIM), lambda i: (0, 0)),
            pl.BlockSpec((1, OUT_DIM), lambda i: (0, 0)),
            pl.BlockSpec((1, OUT_DIM), lambda i: (0, 0)),
            pl.BlockSpec((1, 2), lambda i: (0, 0)),
        ],
        out_specs=[
            pl.BlockSpec((block_rows, OUT_DIM), lambda i: (i, 0)),
            pl.BlockSpec((block_rows, OUT_DIM), lambda i: (i, 0)),
        ],
        out_shape=[
            jax.ShapeDtypeStruct((n, OUT_DIM), jnp.float32),
            jax.ShapeDtypeStruct((n, OUT_DIM), jnp.float32),
        ],
    )(agg0, agg1, h_paper, h_author, Wa0h, ba0.reshape(1, -1),
      ba1.reshape(1, -1), alphas)


def _fold(W, b, rel):
    Wf = jnp.einsum('ihj,hjk->ihk', W.reshape(IN_DIM, N_HEADS, D_K),
                    rel).reshape(IN_DIM, OUT_DIM)
    bf = jnp.einsum('hj,hjk->hk', b.reshape(N_HEADS, D_K), rel).reshape(OUT_DIM)
    return Wf, bf


_SC_MESH = plsc.VectorSubcoreMesh(core_axis_name="c", subcore_axis_name="s")


@functools.partial(
    pl.kernel,
    out_type=[
        jax.ShapeDtypeStruct((E_EDGES, N_HEADS), jnp.float32),
        jax.ShapeDtypeStruct((E_EDGES, N_HEADS), jnp.float32),
        jax.ShapeDtypeStruct((2, 2, NPAD, N_HEADS), jnp.float32),
    ],
    mesh=_SC_MESH,
    compiler_params=pltpu.CompilerParams(use_tc_tiling_on_sc=False,
                                         needs_layout_passes=False),
    scratch_types=[
        pltpu.VMEM((ACHUNK,), jnp.int32),
        pltpu.VMEM((ACHUNK,), jnp.int32),
        pltpu.VMEM((ACHUNK,), jnp.int32),
        pltpu.VMEM((ACHUNK,), jnp.int32),
        pltpu.VMEM((ACHUNK, IN_DIM), jnp.float32),
        pltpu.VMEM((ACHUNK, IN_DIM), jnp.float32),
        pltpu.VMEM((ACHUNK, IN_DIM), jnp.float32),
        pltpu.VMEM((ACHUNK, IN_DIM), jnp.float32),
        pltpu.VMEM((ACHUNK, N_HEADS), jnp.float32),
        pltpu.VMEM((ACHUNK, N_HEADS), jnp.float32),
        pltpu.VMEM_SHARED((NPAD, N_HEADS), jnp.float32),
        pltpu.SemaphoreType.DMA,
        pltpu.SemaphoreType.DMA,
        pltpu.SemaphoreType.DMA,
        pltpu.SemaphoreType.DMA,
    ],
)
def _scores_sc(k0_hbm, q_hbm, k1_hbm, eg0_hbm, eg1_hbm, z8_hbm,
               e0_hbm, e1_hbm, den_hbm,
               srcb0, srcb1, dstb0, dstb1, kbuf0, kbuf1, qbuf0, qbuf1,
               ebuf0, ebuf1, den_sh, sidx0, sidx1, sdat0, sdat1):
    c = lax.axis_index("c")
    s = lax.axis_index("s")
    w = s * 2 + c
    lane16 = lax.iota(jnp.int32, 16)
    g_lo = lax.shift_right_logical(ANGROUP * w, LOG2_NTILE)
    g_hi = lax.shift_right_logical(ANGROUP * (w + 1), LOG2_NTILE)
    nbase = NODES_PER_TILE * s
    srcb = [srcb0, srcb1]
    dstb = [dstb0, dstb1]
    kbuf = [kbuf0, kbuf1]
    qbuf = [qbuf0, qbuf1]
    ebuf = [ebuf0, ebuf1]
    sidx = [sidx0, sidx1]
    sdat = [sdat0, sdat1]

    for ridx, (eg_hbm, k_hbm, e_hbm) in enumerate(
            [(eg0_hbm, k0_hbm, e0_hbm), (eg1_hbm, k1_hbm, e1_hbm)]):
        pltpu.sync_copy(z8_hbm.at[pl.ds(nbase, NODES_PER_TILE)],
                        den_sh.at[pl.ds(nbase, NODES_PER_TILE)])
        plsc.subcore_barrier()

        def issue_idx(g, p):
            eb = g * ACHUNK
            pltpu.async_copy(eg_hbm.at[pl.ds(eb, ACHUNK)], srcb[p], sidx[p])
            pltpu.async_copy(eg_hbm.at[pl.ds(E_EDGES + eb, ACHUNK)],
                             dstb[p], sidx[p])

        def wait_idx(p):
            pltpu.make_async_copy(eg_hbm.at[pl.ds(0, ACHUNK)],
                                  srcb[p], sidx[p]).wait()
            pltpu.make_async_copy(eg_hbm.at[pl.ds(0, ACHUNK)],
                                  dstb[p], sidx[p]).wait()

        def issue_data(p):
            pltpu.async_copy(k_hbm.at[srcb[p]], kbuf[p], sdat[p])
            pltpu.async_copy(q_hbm.at[dstb[p]], qbuf[p], sdat[p])

        def wait_data(p):
            pltpu.make_async_copy(k_hbm.at[srcb[p]], kbuf[p], sdat[p]).wait()
            pltpu.make_async_copy(q_hbm.at[dstb[p]], qbuf[p], sdat[p]).wait()

        def step(g, p):
            @pl.when(g < g_hi)
            def _():
                wait_data(p)

                @pl.when(g + 1 < g_hi)
                def _():
                    wait_idx(1 - p)
                    issue_data(1 - p)

                @pl.loop(0, ACHUNK // 16)
                def _sub(b):
                    lanes = lane16 + 16 * b
                    for h in range(N_HEADS):
                        acc = jnp.zeros((16,), jnp.float32)
                        for d2 in range(D_K):
                            d = jnp.full((16,), D_K * h + d2, jnp.int32)
                            qc = plsc.load_gather(qbuf[p], [lanes, d])
                            kc = plsc.load_gather(kbuf[p], [lanes, d])
                            acc = acc + qc * kc
                        eh = jnp.exp(acc)
                        plsc.store_scatter(
                            ebuf[p], [lanes, jnp.full((16,), h, jnp.int32)],
                            eh)
                eb = g * ACHUNK
                pltpu.sync_copy(ebuf[p], e_hbm.at[pl.ds(eb, ACHUNK)])
                pltpu.sync_copy(ebuf[p], den_sh.at[dstb[p]], add=True)

                @pl.when(g + 2 < g_hi)
                def _():
                    issue_idx(g + 2, p)

        pltpu.sync_copy(eg_hbm.at[pl.ds(g_lo * ACHUNK, ACHUNK)], srcb[0])
        pltpu.sync_copy(eg_hbm.at[pl.ds(E_EDGES + g_lo * ACHUNK, ACHUNK)],
                        dstb[0])
        issue_data(0)

        @pl.when(g_lo + 1 < g_hi)
        def _():
            issue_idx(g_lo + 1, 1)

        npair = lax.shift_right_logical(g_hi - g_lo + 1, 1)

        @pl.loop(0, npair)
        def _pair(t):
            g0 = g_lo + 2 * t
            step(g0, 0)
            step(g0 + 1, 1)

        plsc.subcore_barrier()
        pltpu.sync_copy(den_sh.at[pl.ds(nbase, NODES_PER_TILE)],
                        den_hbm.at[c, ridx, pl.ds(nbase, NODES_PER_TILE)])
        plsc.subcore_barrier()


HALF = 128


@functools.partial(
    pl.kernel,
    out_type=[
        jax.ShapeDtypeStruct((2, NPAD, HALF), jnp.float32),
        jax.ShapeDtypeStruct((2, NPAD, HALF), jnp.float32),
    ],
    mesh=_SC_MESH,
    compiler_params=pltpu.CompilerParams(use_tc_tiling_on_sc=False,
                                         needs_layout_passes=False),
    scratch_types=[
        pltpu.VMEM((CHUNK,), jnp.int32),
        pltpu.VMEM((CHUNK,), jnp.int32),
        pltpu.VMEM((CHUNK,), jnp.int32),
        pltpu.VMEM((CHUNK,), jnp.int32),
        pltpu.VMEM((CHUNK,), jnp.int32),
        pltpu.VMEM((CHUNK,), jnp.int32),
        pltpu.VMEM((CHUNK, HALF), jnp.float32),
        pltpu.VMEM((CHUNK, HALF), jnp.float32),
        pltpu.VMEM((CHUNK, N_HEADS), jnp.float32),
        pltpu.VMEM((CHUNK, N_HEADS), jnp.float32),
        pltpu.VMEM((CHUNK, N_HEADS), jnp.float32),
        pltpu.VMEM((CHUNK, N_HEADS), jnp.float32),
        pltpu.VMEM_SHARED((NPAD, HALF), jnp.float32),
        pltpu.SemaphoreType.DMA,
        pltpu.SemaphoreType.DMA,
        pltpu.SemaphoreType.DMA,
        pltpu.SemaphoreType.DMA,
    ],
)
def _messages_sc(eg0_hbm, eg1_hbm, e0_hbm, e1_hbm, v0_hbm, v1_hbm, den_hbm,
                 zbig_hbm, agg0_hbm, agg1_hbm,
                 srcb0, srcb1, dstb0, dstb1, vidx0, vidx1, vbuf0, vbuf1,
                 ebuf0, ebuf1, d0b, d1b, num_sh,
                 sidx0, sidx1, sdat0, sdat1):
    c = lax.axis_index("c")
    s = lax.axis_index("s")
    g_lo = lax.shift_right_logical(NGROUP * s, LOG2_NSUB)
    g_hi = lax.shift_right_logical(NGROUP * (s + 1), LOG2_NSUB)
    nbase = NODES_PER_TILE * s
    base_h = 4 * c
    srcb = [srcb0, srcb1]
    dstb = [dstb0, dstb1]
    vidxb = [vidx0, vidx1]
    vbuf = [vbuf0, vbuf1]
    ebuf = [ebuf0, ebuf1]
    sidx = [sidx0, sidx1]
    sdat = [sdat0, sdat1]

    for ridx, (eg_hbm, e_hbm, v_hbm, agg_hbm) in enumerate(
            [(eg0_hbm, e0_hbm, v0_hbm, agg0_hbm),
             (eg1_hbm, e1_hbm, v1_hbm, agg1_hbm)]):
        pltpu.sync_copy(zbig_hbm.at[pl.ds(nbase, NODES_PER_TILE)],
                        num_sh.at[pl.ds(nbase, NODES_PER_TILE)])
        plsc.subcore_barrier()

        def issue_idx(g, p):
            eb = g * CHUNK
            pltpu.async_copy(eg_hbm.at[pl.ds(eb, CHUNK)], srcb[p], sidx[p])
            pltpu.async_copy(eg_hbm.at[pl.ds(E_EDGES + eb, CHUNK)],
                             dstb[p], sidx[p])

        def wait_idx(p):
            pltpu.make_async_copy(eg_hbm.at[pl.ds(0, CHUNK)],
                                  srcb[p], sidx[p]).wait()
            pltpu.make_async_copy(eg_hbm.at[pl.ds(0, CHUNK)],
                                  dstb[p], sidx[p]).wait()

        def compute_vidx(p):
            for b in range(CHUNK // 16):
                sl = pl.ds(16 * b, 16)
                vidxb[p][sl] = srcb[p][sl] * 2 + c

        def issue_data(g, p):
            eb = g * CHUNK
            pltpu.async_copy(v_hbm.at[vidxb[p]], vbuf[p], sdat[p])
            pltpu.async_copy(e_hbm.at[pl.ds(eb, CHUNK)], ebuf[p], sdat[p])

        def wait_data(p):
            pltpu.make_async_copy(v_hbm.at[vidxb[p]], vbuf[p], sdat[p]).wait()
            pltpu.make_async_copy(e_hbm.at[pl.ds(0, CHUNK)],
                                  ebuf[p], sdat[p]).wait()

        def step(g, p):
            @pl.when(g < g_hi)
            def _():
                wait_data(p)

                @pl.when(g + 1 < g_hi)
                def _():
                    wait_idx(1 - p)
                    compute_vidx(1 - p)
                    issue_data(g + 1, 1 - p)

                @pl.loop(0, CHUNK)
                def _edge(i):
                    iv = jnp.full((16,), i, jnp.int32)
                    for j2 in range(4):
                        hv = jnp.full((16,), base_h + j2, jnp.int32)
                        sv = plsc.load_gather(ebuf[p], [iv, hv])
                        for half in range(2):
                            off = 32 * j2 + 16 * half
                            vbuf[p][i, pl.ds(off, 16)] = (
                                vbuf[p][i, pl.ds(off, 16)] * sv)
                pltpu.sync_copy(vbuf[p], num_sh.at[dstb[p]], add=True)

                @pl.when(g + 2 < g_hi)
                def _():
                    issue_idx(g + 2, p)

        pltpu.sync_copy(eg_hbm.at[pl.ds(g_lo * CHUNK, CHUNK)], srcb[0])
        pltpu.sync_copy(eg_hbm.at[pl.ds(E_EDGES + g_lo * CHUNK, CHUNK)],
                        dstb[0])
        compute_vidx(0)
        issue_data(g_lo, 0)

        @pl.when(g_lo + 1 < g_hi)
        def _():
            issue_idx(g_lo + 1, 1)

        npair = lax.shift_right_logical(g_hi - g_lo + 1, 1)

        @pl.loop(0, npair)
        def _pair(t):
            g0 = g_lo + 2 * t
            step(g0, 0)
            step(g0 + 1, 1)

        plsc.subcore_barrier()

        @pl.loop(0, NODES_PER_TILE // CHUNK)
        def _drain(t):
            nb = nbase + CHUNK * t
            pltpu.sync_copy(num_sh.at[pl.ds(nb, CHUNK)], vbuf0)
            pltpu.sync_copy(den_hbm.at[0, ridx, pl.ds(nb, CHUNK)], d0b)
            pltpu.sync_copy(den_hbm.at[1, ridx, pl.ds(nb, CHUNK)], d1b)

            @pl.loop(0, CHUNK)
            def _node(n):
                nv = jnp.full((16,), n, jnp.int32)
                for j2 in range(4):
                    hv = jnp.full((16,), base_h + j2, jnp.int32)
                    sden = (plsc.load_gather(d0b, [nv, hv])
                            + plsc.load_gather(d1b, [nv, hv]) + 1e-9)
                    rv = 1.0 / sden
                    for half in range(2):
                        off = 32 * j2 + 16 * half
                        vbuf0[n, pl.ds(off, 16)] = vbuf0[n, pl.ds(off, 16)] * rv
            pltpu.sync_copy(vbuf0, agg_hbm.at[c, pl.ds(nb, CHUNK)])
        plsc.subcore_barrier()


def kernel(h_paper, h_author, edge_writes, edge_cites, Wk, bk, Wv, bv, Wq, bq,
           Wa, ba, rel_att, rel_msg, rel_pri, skip):
    sqrt_dk = math.sqrt(D_K)
    att0 = rel_att[0] * (rel_pri[0][:, None, None] / sqrt_dk)
    att1 = rel_att[1] * (rel_pri[1][:, None, None] / sqrt_dk)
    Wk0, bk0 = _fold(Wk[1], bk[1], att0)
    Wv0, bv0 = _fold(Wv[1], bv[1], rel_msg[0])
    Wk1, bk1 = _fold(Wk[0], bk[0], att1)
    Wv1, bv1 = _fold(Wv[0], bv[0], rel_msg[1])

    K0 = _mm_bias(h_author, Wk0, bk0)
    V0 = _mm_bias(h_author, Wv0, bv0)
    K1 = _mm_bias(h_paper, Wk1, bk1)
    V1 = _mm_bias(h_paper, Wv1, bv1)
    Q = _mm_bias(h_paper, Wq[0], bq[0])

    z8 = jnp.zeros((NPAD, N_HEADS), jnp.float32)
    ew_flat = edge_writes.reshape(-1)
    ec_flat = edge_cites.reshape(-1)
    e0, e1, den = _scores_sc(K0, Q, K1, ew_flat, ec_flat, z8)

    zbig = jnp.zeros((NPAD, HALF), jnp.float32)
    V0f = V0.reshape(N_AUTHOR, 2, HALF).reshape(2 * N_AUTHOR, HALF)
    V1f = V1.reshape(N_PAPER, 2, HALF).reshape(2 * N_PAPER, HALF)
    agg0h, agg1h = _messages_sc(ew_flat, ec_flat, e0, e1, V0f, V1f, den, zbig)
    agg0 = jnp.concatenate([agg0h[0, :N_PAPER], agg0h[1, :N_PAPER]], axis=1)
    agg1 = jnp.concatenate([agg1h[0, :N_PAPER], agg1h[1, :N_PAPER]], axis=1)

    out_p, out_a = _final(agg0, agg1, h_paper, h_author, 0.5 * Wa[0],
                          ba[0], ba[1], skip)
    return (out_p, out_a)


# speedup vs baseline: 14.8940x; 1.3658x over previous
import math
import functools
import jax
import jax.numpy as jnp
from jax import lax
from jax.experimental import pallas as pl
from jax.experimental.pallas import tpu as pltpu
from jax.experimental.pallas import tpu_sc as plsc

N_PAPER = 10000
N_AUTHOR = 10000
E_EDGES = 160000
IN_DIM = 256
OUT_DIM = 256
N_HEADS = 8
D_K = OUT_DIM // N_HEADS
NTILE = 32
CHUNK = 128
NGROUP = E_EDGES // CHUNK
ACHUNK = 128
ANGROUP = E_EDGES // ACHUNK
PK = IN_DIM // 2
LOG2_NTILE = 5
LOG2_NSUB = 4
NPAD = 10240
NODES_PER_TILE = NPAD // 16


def _mm_bias_kernel(x_ref, w_ref, b_ref, o_ref):
    o_ref[...] = jnp.dot(x_ref[...], w_ref[...],
                         preferred_element_type=jnp.float32) + b_ref[...]


def _mm_bias(x, w, b, block_rows=2000):
    n = x.shape[0]
    return pl.pallas_call(
        _mm_bias_kernel,
        grid=(n // block_rows,),
        in_specs=[
            pl.BlockSpec((block_rows, x.shape[1]), lambda i: (i, 0)),
            pl.BlockSpec((w.shape[0], w.shape[1]), lambda i: (0, 0)),
            pl.BlockSpec((1, w.shape[1]), lambda i: (0, 0)),
        ],
        out_specs=pl.BlockSpec((block_rows, w.shape[1]), lambda i: (i, 0)),
        out_shape=jax.ShapeDtypeStruct((n, w.shape[1]), jnp.float32),
    )(x, w, b.reshape(1, -1))


def _final_kernel(a0_ref, a1_ref, hp_ref, ha_ref, wa0_ref, ba0_ref, ba1_ref,
                  sk_ref, op_ref, oa_ref):
    alpha0 = sk_ref[0, 0]
    alpha1 = sk_ref[0, 1]
    agg = a0_ref[...] + a1_ref[...]
    t = jnp.dot(agg, wa0_ref[...],
                preferred_element_type=jnp.float32) + ba0_ref[...]
    op_ref[...] = t * alpha0 + hp_ref[...] * (1.0 - alpha0)
    oa_ref[...] = ba1_ref[...] * alpha1 + ha_ref[...] * (1.0 - alpha1)


def _final(agg0, agg1, h_paper, h_author, Wa0h, ba0, ba1, skip,
           block_rows=2000):
    n = N_PAPER
    alphas = jax.nn.sigmoid(skip).reshape(1, 2)
    return pl.pallas_call(
        _final_kernel,
        grid=(n // block_rows,),
        in_specs=[
            pl.BlockSpec((block_rows, OUT_DIM), lambda i: (i, 0)),
            pl.BlockSpec((block_rows, OUT_DIM), lambda i: (i, 0)),
            pl.BlockSpec((block_rows, IN_DIM), lambda i: (i, 0)),
            pl.BlockSpec((block_rows, IN_DIM), lambda i: (i, 0)),
            pl.BlockSpec((OUT_DIM, OUT_DIM), lambda i: (0, 0)),
            pl.BlockSpec((1, OUT_DIM), lambda i: (0, 0)),
            pl.BlockSpec((1, OUT_DIM), lambda i: (0, 0)),
            pl.BlockSpec((1, 2), lambda i: (0, 0)),
        ],
        out_specs=[
            pl.BlockSpec((block_rows, OUT_DIM), lambda i: (i, 0)),
            pl.BlockSpec((block_rows, OUT_DIM), lambda i: (i, 0)),
        ],
        out_shape=[
            jax.ShapeDtypeStruct((n, OUT_DIM), jnp.float32),
            jax.ShapeDtypeStruct((n, OUT_DIM), jnp.float32),
        ],
    )(agg0, agg1, h_paper, h_author, Wa0h, ba0.reshape(1, -1),
      ba1.reshape(1, -1), alphas)


def _fold(W, b, rel):
    Wf = jnp.einsum('ihj,hjk->ihk', W.reshape(IN_DIM, N_HEADS, D_K),
                    rel).reshape(IN_DIM, OUT_DIM)
    bf = jnp.einsum('hj,hjk->hk', b.reshape(N_HEADS, D_K), rel).reshape(OUT_DIM)
    return Wf, bf


_SC_MESH = plsc.VectorSubcoreMesh(core_axis_name="c", subcore_axis_name="s")


@functools.partial(
    pl.kernel,
    out_type=[
        jax.ShapeDtypeStruct((E_EDGES, N_HEADS), jnp.float32),
        jax.ShapeDtypeStruct((E_EDGES, N_HEADS), jnp.float32),
        jax.ShapeDtypeStruct((2, 2, NPAD, N_HEADS), jnp.float32),
    ],
    mesh=_SC_MESH,
    compiler_params=pltpu.CompilerParams(use_tc_tiling_on_sc=False,
                                         needs_layout_passes=False),
    scratch_types=[
        pltpu.VMEM((ACHUNK,), jnp.int32),
        pltpu.VMEM((ACHUNK,), jnp.int32),
        pltpu.VMEM((ACHUNK,), jnp.int32),
        pltpu.VMEM((ACHUNK,), jnp.int32),
        pltpu.VMEM((ACHUNK, PK), jnp.int32),
        pltpu.VMEM((ACHUNK, PK), jnp.int32),
        pltpu.VMEM((ACHUNK, PK), jnp.int32),
        pltpu.VMEM((ACHUNK, PK), jnp.int32),
        pltpu.VMEM((ACHUNK, N_HEADS), jnp.float32),
        pltpu.VMEM((ACHUNK, N_HEADS), jnp.float32),
        pltpu.VMEM_SHARED((NPAD, N_HEADS), jnp.float32),
        pltpu.SemaphoreType.DMA,
        pltpu.SemaphoreType.DMA,
        pltpu.SemaphoreType.DMA,
        pltpu.SemaphoreType.DMA,
    ],
)
def _scores_sc(k0_hbm, q_hbm, k1_hbm, eg0_hbm, eg1_hbm, z8_hbm,
               e0_hbm, e1_hbm, den_hbm,
               srcb0, srcb1, dstb0, dstb1, kbuf0, kbuf1, qbuf0, qbuf1,
               ebuf0, ebuf1, den_sh, sidx0, sidx1, sdat0, sdat1):
    c = lax.axis_index("c")
    s = lax.axis_index("s")
    w = s * 2 + c
    lane16 = lax.iota(jnp.int32, 16)
    g_lo = lax.shift_right_logical(ANGROUP * w, LOG2_NTILE)
    g_hi = lax.shift_right_logical(ANGROUP * (w + 1), LOG2_NTILE)
    nbase = NODES_PER_TILE * s
    srcb = [srcb0, srcb1]
    dstb = [dstb0, dstb1]
    kbuf = [kbuf0, kbuf1]
    qbuf = [qbuf0, qbuf1]
    ebuf = [ebuf0, ebuf1]
    sidx = [sidx0, sidx1]
    sdat = [sdat0, sdat1]

    for ridx, (eg_hbm, k_hbm, e_hbm) in enumerate(
            [(eg0_hbm, k0_hbm, e0_hbm), (eg1_hbm, k1_hbm, e1_hbm)]):
        pltpu.sync_copy(z8_hbm.at[pl.ds(nbase, NODES_PER_TILE)],
                        den_sh.at[pl.ds(nbase, NODES_PER_TILE)])
        plsc.subcore_barrier()

        def issue_idx(g, p):
            eb = g * ACHUNK
            pltpu.async_copy(eg_hbm.at[pl.ds(eb, ACHUNK)], srcb[p], sidx[p])
            pltpu.async_copy(eg_hbm.at[pl.ds(E_EDGES + eb, ACHUNK)],
                             dstb[p], sidx[p])

        def wait_idx(p):
            pltpu.make_async_copy(eg_hbm.at[pl.ds(0, ACHUNK)],
                                  srcb[p], sidx[p]).wait()
            pltpu.make_async_copy(eg_hbm.at[pl.ds(0, ACHUNK)],
                                  dstb[p], sidx[p]).wait()

        def issue_data(p):
            pltpu.async_copy(k_hbm.at[srcb[p]], kbuf[p], sdat[p])
            pltpu.async_copy(q_hbm.at[dstb[p]], qbuf[p], sdat[p])

        def wait_data(p):
            pltpu.make_async_copy(k_hbm.at[srcb[p]], kbuf[p], sdat[p]).wait()
            pltpu.make_async_copy(q_hbm.at[dstb[p]], qbuf[p], sdat[p]).wait()

        def step(g, p):
            @pl.when(g < g_hi)
            def _():
                wait_data(p)

                @pl.when(g + 1 < g_hi)
                def _():
                    wait_idx(1 - p)
                    issue_data(1 - p)

                @pl.loop(0, ACHUNK // 16)
                def _sub(b):
                    lanes = lane16 + 16 * b
                    for h in range(N_HEADS):
                        acc = jnp.zeros((16,), jnp.float32)
                        for dp in range(D_K // 2):
                            d = jnp.full((16,), (D_K // 2) * h + dp, jnp.int32)
                            qc = plsc.load_gather(qbuf[p], [lanes, d])
                            kc = plsc.load_gather(kbuf[p], [lanes, d])
                            ql, qh = plsc.unpack(
                                plsc.bitcast(qc, jnp.bfloat16),
                                format=plsc.PackFormat.INTERLEAVED)
                            kl, kh = plsc.unpack(
                                plsc.bitcast(kc, jnp.bfloat16),
                                format=plsc.PackFormat.INTERLEAVED)
                            acc = acc + ql * kl + qh * kh
                        eh = jnp.exp(acc)
                        plsc.store_scatter(
                            ebuf[p], [lanes, jnp.full((16,), h, jnp.int32)],
                            eh)
                eb = g * ACHUNK
                pltpu.sync_copy(ebuf[p], e_hbm.at[pl.ds(eb, ACHUNK)])
                pltpu.sync_copy(ebuf[p], den_sh.at[dstb[p]], add=True)

                @pl.when(g + 2 < g_hi)
                def _():
                    issue_idx(g + 2, p)

        pltpu.sync_copy(eg_hbm.at[pl.ds(g_lo * ACHUNK, ACHUNK)], srcb[0])
        pltpu.sync_copy(eg_hbm.at[pl.ds(E_EDGES + g_lo * ACHUNK, ACHUNK)],
                        dstb[0])
        issue_data(0)

        @pl.when(g_lo + 1 < g_hi)
        def _():
            issue_idx(g_lo + 1, 1)

        npair = lax.shift_right_logical(g_hi - g_lo + 1, 1)

        @pl.loop(0, npair)
        def _pair(t):
            g0 = g_lo + 2 * t
            step(g0, 0)
            step(g0 + 1, 1)

        plsc.subcore_barrier()
        pltpu.sync_copy(den_sh.at[pl.ds(nbase, NODES_PER_TILE)],
                        den_hbm.at[c, ridx, pl.ds(nbase, NODES_PER_TILE)])
        plsc.subcore_barrier()


HALF = 128


@functools.partial(
    pl.kernel,
    out_type=[
        jax.ShapeDtypeStruct((2, NPAD, HALF), jnp.float32),
        jax.ShapeDtypeStruct((2, NPAD, HALF), jnp.float32),
    ],
    mesh=_SC_MESH,
    compiler_params=pltpu.CompilerParams(use_tc_tiling_on_sc=False,
                                         needs_layout_passes=False),
    scratch_types=[
        pltpu.VMEM((CHUNK,), jnp.int32),
        pltpu.VMEM((CHUNK,), jnp.int32),
        pltpu.VMEM((CHUNK,), jnp.int32),
        pltpu.VMEM((CHUNK,), jnp.int32),
        pltpu.VMEM((CHUNK,), jnp.int32),
        pltpu.VMEM((CHUNK,), jnp.int32),
        pltpu.VMEM((CHUNK, HALF), jnp.float32),
        pltpu.VMEM((CHUNK, HALF), jnp.float32),
        pltpu.VMEM((CHUNK, N_HEADS), jnp.float32),
        pltpu.VMEM((CHUNK, N_HEADS), jnp.float32),
        pltpu.VMEM((CHUNK, N_HEADS), jnp.float32),
        pltpu.VMEM((CHUNK, N_HEADS), jnp.float32),
        pltpu.VMEM_SHARED((NPAD, HALF), jnp.float32),
        pltpu.SemaphoreType.DMA,
        pltpu.SemaphoreType.DMA,
        pltpu.SemaphoreType.DMA,
        pltpu.SemaphoreType.DMA,
    ],
)
def _messages_sc(eg0_hbm, eg1_hbm, e0_hbm, e1_hbm, v0_hbm, v1_hbm, den_hbm,
                 zbig_hbm, agg0_hbm, agg1_hbm,
                 srcb0, srcb1, dstb0, dstb1, vidx0, vidx1, vbuf0, vbuf1,
                 ebuf0, ebuf1, d0b, d1b, num_sh,
                 sidx0, sidx1, sdat0, sdat1):
    c = lax.axis_index("c")
    s = lax.axis_index("s")
    g_lo = lax.shift_right_logical(NGROUP * s, LOG2_NSUB)
    g_hi = lax.shift_right_logical(NGROUP * (s + 1), LOG2_NSUB)
    nbase = NODES_PER_TILE * s
    base_h = 4 * c
    srcb = [srcb0, srcb1]
    dstb = [dstb0, dstb1]
    vidxb = [vidx0, vidx1]
    vbuf = [vbuf0, vbuf1]
    ebuf = [ebuf0, ebuf1]
    sidx = [sidx0, sidx1]
    sdat = [sdat0, sdat1]

    for ridx, (eg_hbm, e_hbm, v_hbm, agg_hbm) in enumerate(
            [(eg0_hbm, e0_hbm, v0_hbm, agg0_hbm),
             (eg1_hbm, e1_hbm, v1_hbm, agg1_hbm)]):
        pltpu.sync_copy(zbig_hbm.at[pl.ds(nbase, NODES_PER_TILE)],
                        num_sh.at[pl.ds(nbase, NODES_PER_TILE)])
        plsc.subcore_barrier()

        def issue_idx(g, p):
            eb = g * CHUNK
            pltpu.async_copy(eg_hbm.at[pl.ds(eb, CHUNK)], srcb[p], sidx[p])
            pltpu.async_copy(eg_hbm.at[pl.ds(E_EDGES + eb, CHUNK)],
                             dstb[p], sidx[p])

        def wait_idx(p):
            pltpu.make_async_copy(eg_hbm.at[pl.ds(0, CHUNK)],
                                  srcb[p], sidx[p]).wait()
            pltpu.make_async_copy(eg_hbm.at[pl.ds(0, CHUNK)],
                                  dstb[p], sidx[p]).wait()

        def compute_vidx(p):
            for b in range(CHUNK // 16):
                sl = pl.ds(16 * b, 16)
                vidxb[p][sl] = srcb[p][sl] * 2 + c

        def issue_data(g, p):
            eb = g * CHUNK
            pltpu.async_copy(v_hbm.at[vidxb[p]], vbuf[p], sdat[p])
            pltpu.async_copy(e_hbm.at[pl.ds(eb, CHUNK)], ebuf[p], sdat[p])

        def wait_data(p):
            pltpu.make_async_copy(v_hbm.at[vidxb[p]], vbuf[p], sdat[p]).wait()
            pltpu.make_async_copy(e_hbm.at[pl.ds(0, CHUNK)],
                                  ebuf[p], sdat[p]).wait()

        def step(g, p):
            @pl.when(g < g_hi)
            def _():
                wait_data(p)

                @pl.when(g + 1 < g_hi)
                def _():
                    wait_idx(1 - p)
                    compute_vidx(1 - p)
                    issue_data(g + 1, 1 - p)

                @pl.loop(0, CHUNK)
                def _edge(i):
                    iv = jnp.full((16,), i, jnp.int32)
                    for j2 in range(4):
                        hv = jnp.full((16,), base_h + j2, jnp.int32)
                        sv = plsc.load_gather(ebuf[p], [iv, hv])
                        for half in range(2):
                            off = 32 * j2 + 16 * half
                            vbuf[p][i, pl.ds(off, 16)] = (
                                vbuf[p][i, pl.ds(off, 16)] * sv)
                pltpu.sync_copy(vbuf[p], num_sh.at[dstb[p]], add=True)

                @pl.when(g + 2 < g_hi)
                def _():
                    issue_idx(g + 2, p)

        pltpu.sync_copy(eg_hbm.at[pl.ds(g_lo * CHUNK, CHUNK)], srcb[0])
        pltpu.sync_copy(eg_hbm.at[pl.ds(E_EDGES + g_lo * CHUNK, CHUNK)],
                        dstb[0])
        compute_vidx(0)
        issue_data(g_lo, 0)

        @pl.when(g_lo + 1 < g_hi)
        def _():
            issue_idx(g_lo + 1, 1)

        npair = lax.shift_right_logical(g_hi - g_lo + 1, 1)

        @pl.loop(0, npair)
        def _pair(t):
            g0 = g_lo + 2 * t
            step(g0, 0)
            step(g0 + 1, 1)

        plsc.subcore_barrier()

        @pl.loop(0, NODES_PER_TILE // CHUNK)
        def _drain(t):
            nb = nbase + CHUNK * t
            pltpu.sync_copy(num_sh.at[pl.ds(nb, CHUNK)], vbuf0)
            pltpu.sync_copy(den_hbm.at[0, ridx, pl.ds(nb, CHUNK)], d0b)
            pltpu.sync_copy(den_hbm.at[1, ridx, pl.ds(nb, CHUNK)], d1b)

            @pl.loop(0, CHUNK)
            def _node(n):
                nv = jnp.full((16,), n, jnp.int32)
                for j2 in range(4):
                    hv = jnp.full((16,), base_h + j2, jnp.int32)
                    sden = (plsc.load_gather(d0b, [nv, hv])
                            + plsc.load_gather(d1b, [nv, hv]) + 1e-9)
                    rv = 1.0 / sden
                    for half in range(2):
                        off = 32 * j2 + 16 * half
                        vbuf0[n, pl.ds(off, 16)] = vbuf0[n, pl.ds(off, 16)] * rv
            pltpu.sync_copy(vbuf0, agg_hbm.at[c, pl.ds(nb, CHUNK)])
        plsc.subcore_barrier()


def kernel(h_paper, h_author, edge_writes, edge_cites, Wk, bk, Wv, bv, Wq, bq,
           Wa, ba, rel_att, rel_msg, rel_pri, skip):
    sqrt_dk = math.sqrt(D_K)
    att0 = rel_att[0] * (rel_pri[0][:, None, None] / sqrt_dk)
    att1 = rel_att[1] * (rel_pri[1][:, None, None] / sqrt_dk)
    Wk0, bk0 = _fold(Wk[1], bk[1], att0)
    Wv0, bv0 = _fold(Wv[1], bv[1], rel_msg[0])
    Wk1, bk1 = _fold(Wk[0], bk[0], att1)
    Wv1, bv1 = _fold(Wv[0], bv[0], rel_msg[1])

    K0 = _mm_bias(h_author, Wk0, bk0)
    V0 = _mm_bias(h_author, Wv0, bv0)
    K1 = _mm_bias(h_paper, Wk1, bk1)
    V1 = _mm_bias(h_paper, Wv1, bv1)
    Q = _mm_bias(h_paper, Wq[0], bq[0])

    def pack_bf16(x):
        xb = x.astype(jnp.bfloat16).reshape(x.shape[0], PK, 2)
        return jax.lax.bitcast_convert_type(xb, jnp.int32)

    z8 = jnp.zeros((NPAD, N_HEADS), jnp.float32)
    ew_flat = edge_writes.reshape(-1)
    ec_flat = edge_cites.reshape(-1)
    e0, e1, den = _scores_sc(pack_bf16(K0), pack_bf16(Q), pack_bf16(K1),
                             ew_flat, ec_flat, z8)

    zbig = jnp.zeros((NPAD, HALF), jnp.float32)
    V0f = V0.reshape(N_AUTHOR, 2, HALF).reshape(2 * N_AUTHOR, HALF)
    V1f = V1.reshape(N_PAPER, 2, HALF).reshape(2 * N_PAPER, HALF)
    agg0h, agg1h = _messages_sc(ew_flat, ec_flat, e0, e1, V0f, V1f, den, zbig)
    agg0 = jnp.concatenate([agg0h[0, :N_PAPER], agg0h[1, :N_PAPER]], axis=1)
    agg1 = jnp.concatenate([agg1h[0, :N_PAPER], agg1h[1, :N_PAPER]], axis=1)

    out_p, out_a = _final(agg0, agg1, h_paper, h_author, 0.5 * Wa[0],
                          ba[0], ba[1], skip)
    return (out_p, out_a)
